# baseline (speedup 1.0000x reference)
import jax
import jax.numpy as jnp
from jax import lax
from jax.experimental import pallas as pl
from jax.experimental.pallas import tpu as pltpu
from jax.experimental.pallas import tpu_sc as plsc

NC = 2
NS = 16
NW = NC * NS

BATCH = 64
N_CLAUSE = 100000
N_SAT = 3
N_VARS = 25000
NV2 = 2 * N_VARS

CH = 2048
SPLITS = (100000,)
B_PER_W = BATCH // NW


def _make_sc_body(ncs):
    nfull = ncs // CH
    tail = ncs - nfull * CH
    return lambda *args: _sc_body_impl(nfull, tail, *args)


def _sc_body_impl(NFULL, TAIL, v_hbm, idx_hbm, out_hbm,
                  v_buf, idx_buf, out_buf, sem):
    wid = lax.axis_index("s") * NC + lax.axis_index("c")

    def start_in(b, c0, slot, cw):
        pltpu.async_copy(idx_hbm.at[:, pl.ds(b, 1), pl.ds(c0, cw)],
                         idx_buf.at[slot, :, :, pl.ds(0, cw)], sem)

    def wait_in(b, c0, slot, cw):
        pltpu.make_async_copy(idx_hbm.at[:, pl.ds(b, 1), pl.ds(c0, cw)],
                              idx_buf.at[slot, :, :, pl.ds(0, cw)], sem).wait()

    def compute(slot, c0, nc16, cw, b):
        def body(i, carry):
            cb = i * 16
            i0 = idx_buf[slot, 0, 0, pl.ds(cb, 16)]
            i1 = idx_buf[slot, 1, 0, pl.ds(cb, 16)]
            i2 = idx_buf[slot, 2, 0, pl.ds(cb, 16)]
            v0 = plsc.load_gather(v_buf.at[0], [i0])
            v1 = plsc.load_gather(v_buf.at[0], [i1])
            v2 = plsc.load_gather(v_buf.at[0], [i2])
            mx = jnp.maximum(jnp.maximum(v0, v1), v2)
            out_buf[0, pl.ds(cb, 16)] = (1.0 - mx) * 0.5
            return carry

        lax.fori_loop(0, nc16, body, 0)
        pltpu.sync_copy(out_buf.at[:, pl.ds(0, cw)],
                        out_hbm.at[pl.ds(b, 1), pl.ds(c0, cw)])

    for k in range(B_PER_W):
        b = wid * B_PER_W + k
        pltpu.sync_copy(v_hbm.at[pl.ds(b * NV2, NV2)], v_buf.at[0])
        start_in(b, 0, 0, CH)

        def loop_body(ci, carry):
            slot = ci % 2
            wait_in(b, ci * CH, slot, CH)
            pl.when(ci < NFULL - 1)(
                lambda: start_in(b, (ci + 1) * CH, 1 - slot, CH))
            pl.when(ci == NFULL - 1)(
                lambda: start_in(b, NFULL * CH, 1 - slot, TAIL))
            compute(slot, ci * CH, CH // 16, CH, b)
            return carry

        lax.fori_loop(0, NFULL, loop_body, 0)
        tslot = NFULL % 2
        wait_in(b, NFULL * CH, tslot, TAIL)
        compute(tslot, NFULL * CH, TAIL // 16, TAIL, b)


@jax.jit
def kernel(v, input_idx, input_sign):
    comb = input_idx.astype(jnp.int32) + jnp.where(
        input_sign < 0, jnp.int32(N_VARS), jnp.int32(0))
    idx = jnp.transpose(comb, (2, 0, 1))
    v2f = jnp.concatenate([v, -v], axis=1).reshape(BATCH * NV2)
    outs = []
    c0 = 0
    for ncs in SPLITS:
        run = pl.kernel(
            _make_sc_body(ncs),
            out_type=jax.ShapeDtypeStruct((BATCH, ncs), jnp.float32),
            mesh=plsc.VectorSubcoreMesh(core_axis_name="c", subcore_axis_name="s"),
            compiler_params=pltpu.CompilerParams(
                needs_layout_passes=False, use_tc_tiling_on_sc=False),
            scratch_types=[
                pltpu.VMEM((1, NV2), jnp.float32),
                pltpu.VMEM((2, N_SAT, 1, CH), jnp.int32),
                pltpu.VMEM((1, CH), jnp.float32),
                pltpu.SemaphoreType.DMA,
            ],
        )
        outs.append(run(v2f, lax.slice_in_dim(idx, c0, c0 + ncs, axis=2)))
        c0 += ncs
    if len(outs) == 1:
        return outs[0]
    return jnp.concatenate(outs, axis=1)


# speedup vs baseline: 1.0656x; 1.0656x over previous
import jax
import jax.numpy as jnp
from jax import lax
from jax.experimental import pallas as pl
from jax.experimental.pallas import tpu as pltpu
from jax.experimental.pallas import tpu_sc as plsc

NC = 2
NS = 16
NW = NC * NS

BATCH = 64
N_CLAUSE = 100000
N_SAT = 3
N_VARS = 25000
NV2 = 2 * N_VARS

CH = 2048
SPLITS = (25088, 25088, 25088, 24736)
B_PER_W = BATCH // NW


def _make_sc_body(ncs):
    nfull = ncs // CH
    tail = ncs - nfull * CH
    return lambda *args: _sc_body_impl(nfull, tail, *args)


def _sc_body_impl(NFULL, TAIL, v_hbm, idx_hbm, out_hbm,
                  v_buf, idx_buf, out_buf, sem):
    wid = lax.axis_index("s") * NC + lax.axis_index("c")

    def start_in(b, c0, slot, cw):
        pltpu.async_copy(idx_hbm.at[:, pl.ds(b, 1), pl.ds(c0, cw)],
                         idx_buf.at[slot, :, :, pl.ds(0, cw)], sem)

    def wait_in(b, c0, slot, cw):
        pltpu.make_async_copy(idx_hbm.at[:, pl.ds(b, 1), pl.ds(c0, cw)],
                              idx_buf.at[slot, :, :, pl.ds(0, cw)], sem).wait()

    def compute(slot, c0, nc16, cw, b):
        def body(i, carry):
            cb = i * 16
            i0 = idx_buf[slot, 0, 0, pl.ds(cb, 16)]
            i1 = idx_buf[slot, 1, 0, pl.ds(cb, 16)]
            i2 = idx_buf[slot, 2, 0, pl.ds(cb, 16)]
            v0 = plsc.load_gather(v_buf.at[0], [i0])
            v1 = plsc.load_gather(v_buf.at[0], [i1])
            v2 = plsc.load_gather(v_buf.at[0], [i2])
            mx = jnp.maximum(jnp.maximum(v0, v1), v2)
            out_buf[0, pl.ds(cb, 16)] = (1.0 - mx) * 0.5
            return carry

        lax.fori_loop(0, nc16, body, 0)
        pltpu.sync_copy(out_buf.at[:, pl.ds(0, cw)],
                        out_hbm.at[pl.ds(b, 1), pl.ds(c0, cw)])

    for k in range(B_PER_W):
        b = wid * B_PER_W + k
        pltpu.sync_copy(v_hbm.at[pl.ds(b * NV2, NV2)], v_buf.at[0])
        start_in(b, 0, 0, CH)

        def loop_body(ci, carry):
            slot = ci % 2
            wait_in(b, ci * CH, slot, CH)
            pl.when(ci < NFULL - 1)(
                lambda: start_in(b, (ci + 1) * CH, 1 - slot, CH))
            pl.when(ci == NFULL - 1)(
                lambda: start_in(b, NFULL * CH, 1 - slot, TAIL))
            compute(slot, ci * CH, CH // 16, CH, b)
            return carry

        lax.fori_loop(0, NFULL, loop_body, 0)
        tslot = NFULL % 2
        wait_in(b, NFULL * CH, tslot, TAIL)
        compute(tslot, NFULL * CH, TAIL // 16, TAIL, b)


@jax.jit
def kernel(v, input_idx, input_sign):
    comb = input_idx.astype(jnp.int32) + jnp.where(
        input_sign < 0, jnp.int32(N_VARS), jnp.int32(0))
    idx = jnp.transpose(comb, (2, 0, 1))
    v2f = jnp.concatenate([v, -v], axis=1).reshape(BATCH * NV2)
    outs = []
    c0 = 0
    for ncs in SPLITS:
        run = pl.kernel(
            _make_sc_body(ncs),
            out_type=jax.ShapeDtypeStruct((BATCH, ncs), jnp.float32),
            mesh=plsc.VectorSubcoreMesh(core_axis_name="c", subcore_axis_name="s"),
            compiler_params=pltpu.CompilerParams(
                needs_layout_passes=False, use_tc_tiling_on_sc=False),
            scratch_types=[
                pltpu.VMEM((1, NV2), jnp.float32),
                pltpu.VMEM((2, N_SAT, 1, CH), jnp.int32),
                pltpu.VMEM((1, CH), jnp.float32),
                pltpu.SemaphoreType.DMA,
            ],
        )
        outs.append(run(v2f, lax.slice_in_dim(idx, c0, c0 + ncs, axis=2)))
        c0 += ncs
    if len(outs) == 1:
        return outs[0]
    return jnp.concatenate(outs, axis=1)


# speedup vs baseline: 1.0755x; 1.0093x over previous
import jax
import jax.numpy as jnp
from jax import lax
from jax.experimental import pallas as pl
from jax.experimental.pallas import tpu as pltpu
from jax.experimental.pallas import tpu_sc as plsc

NC = 2
NS = 16
NW = NC * NS

BATCH = 64
N_CLAUSE = 100000
N_SAT = 3
N_VARS = 25000
NV2 = 2 * N_VARS

CH = 4096
SPLITS = (25088, 25088, 25088, 24736)
B_PER_W = BATCH // NW


def _make_sc_body(ncs):
    nfull = ncs // CH
    tail = ncs - nfull * CH
    return lambda *args: _sc_body_impl(nfull, tail, *args)


def _sc_body_impl(NFULL, TAIL, v_hbm, idx_hbm, out_hbm,
                  v_buf, idx_buf, out_buf, sem):
    wid = lax.axis_index("s") * NC + lax.axis_index("c")

    def start_in(b, c0, slot, cw):
        pltpu.async_copy(idx_hbm.at[:, pl.ds(b, 1), pl.ds(c0, cw)],
                         idx_buf.at[slot, :, :, pl.ds(0, cw)], sem)

    def wait_in(b, c0, slot, cw):
        pltpu.make_async_copy(idx_hbm.at[:, pl.ds(b, 1), pl.ds(c0, cw)],
                              idx_buf.at[slot, :, :, pl.ds(0, cw)], sem).wait()

    def compute(slot, c0, nc16, cw, b):
        def body(i, carry):
            cb = i * 16
            i0 = idx_buf[slot, 0, 0, pl.ds(cb, 16)]
            i1 = idx_buf[slot, 1, 0, pl.ds(cb, 16)]
            i2 = idx_buf[slot, 2, 0, pl.ds(cb, 16)]
            v0 = plsc.load_gather(v_buf.at[0], [i0])
            v1 = plsc.load_gather(v_buf.at[0], [i1])
            v2 = plsc.load_gather(v_buf.at[0], [i2])
            mx = jnp.maximum(jnp.maximum(v0, v1), v2)
            out_buf[0, pl.ds(cb, 16)] = (1.0 - mx) * 0.5
            return carry

        lax.fori_loop(0, nc16, body, 0)
        pltpu.sync_copy(out_buf.at[:, pl.ds(0, cw)],
                        out_hbm.at[pl.ds(b, 1), pl.ds(c0, cw)])

    for k in range(B_PER_W):
        b = wid * B_PER_W + k
        pltpu.sync_copy(v_hbm.at[pl.ds(b * NV2, NV2)], v_buf.at[0])
        start_in(b, 0, 0, CH)

        def loop_body(ci, carry):
            slot = ci % 2
            wait_in(b, ci * CH, slot, CH)
            pl.when(ci < NFULL - 1)(
                lambda: start_in(b, (ci + 1) * CH, 1 - slot, CH))
            pl.when(ci == NFULL - 1)(
                lambda: start_in(b, NFULL * CH, 1 - slot, TAIL))
            compute(slot, ci * CH, CH // 16, CH, b)
            return carry

        lax.fori_loop(0, NFULL, loop_body, 0)
        tslot = NFULL % 2
        wait_in(b, NFULL * CH, tslot, TAIL)
        compute(tslot, NFULL * CH, TAIL // 16, TAIL, b)


@jax.jit
def kernel(v, input_idx, input_sign):
    comb = input_idx.astype(jnp.int32) + jnp.where(
        input_sign < 0, jnp.int32(N_VARS), jnp.int32(0))
    idx = jnp.transpose(comb, (2, 0, 1))
    v2f = jnp.concatenate([v, -v], axis=1).reshape(BATCH * NV2)
    outs = []
    c0 = 0
    for ncs in SPLITS:
        run = pl.kernel(
            _make_sc_body(ncs),
            out_type=jax.ShapeDtypeStruct((BATCH, ncs), jnp.float32),
            mesh=plsc.VectorSubcoreMesh(core_axis_name="c", subcore_axis_name="s"),
            compiler_params=pltpu.CompilerParams(
                needs_layout_passes=False, use_tc_tiling_on_sc=False),
            scratch_types=[
                pltpu.VMEM((1, NV2), jnp.float32),
                pltpu.VMEM((2, N_SAT, 1, CH), jnp.int32),
                pltpu.VMEM((1, CH), jnp.float32),
                pltpu.SemaphoreType.DMA,
            ],
        )
        outs.append(run(v2f, lax.slice_in_dim(idx, c0, c0 + ncs, axis=2)))
        c0 += ncs
    if len(outs) == 1:
        return outs[0]
    return jnp.concatenate(outs, axis=1)
